# baseline (device time: 56970 ns/iter reference)
import jax
import jax.numpy as jnp
from jax import lax
from jax.experimental import pallas as pl
from jax.experimental.pallas import tpu as pltpu

N_DEV = 16


def kernel(x, w_mat):
    m_per, k = x.shape
    _, n_per = w_mat.shape

    def body(x_ref, w_ref, out_ref, comm_ref, send_sems, recv_sems):
        my = lax.axis_index("i")
        left = lax.rem(my + (N_DEV - 1), N_DEV)
        right = lax.rem(my + 1, N_DEV)

        barrier_sem = pltpu.get_barrier_semaphore()
        pl.semaphore_signal(
            barrier_sem, inc=1,
            device_id=(left,), device_id_type=pl.DeviceIdType.MESH,
        )
        pl.semaphore_wait(barrier_sem, 1)

        w_bf16 = w_ref[...].astype(jnp.bfloat16)

        def store_block(origin, chunk):
            y = jnp.dot(chunk, w_bf16, preferred_element_type=jnp.float32)
            out_ref[pl.ds(origin * m_per, m_per), :] = jnp.maximum(y, 0.0)

        comm_ref[0] = x_ref[...].astype(comm_ref.dtype)
        store_block(my, comm_ref[0])

        for h in range(N_DEV - 1):
            rdma = pltpu.make_async_remote_copy(
                src_ref=comm_ref.at[h],
                dst_ref=comm_ref.at[h + 1],
                send_sem=send_sems.at[h],
                recv_sem=recv_sems.at[h],
                device_id=(right,),
                device_id_type=pl.DeviceIdType.MESH,
            )
            rdma.start()
            rdma.wait()
            origin = lax.rem(my + (N_DEV - 1 - h), N_DEV)
            store_block(origin, comm_ref[h + 1])

    return pl.pallas_call(
        body,
        out_shape=jax.ShapeDtypeStruct((N_DEV * m_per, n_per), jnp.float32),
        in_specs=[
            pl.BlockSpec(memory_space=pltpu.VMEM),
            pl.BlockSpec(memory_space=pltpu.VMEM),
        ],
        out_specs=pl.BlockSpec(memory_space=pltpu.VMEM),
        scratch_shapes=[
            pltpu.VMEM((N_DEV, m_per, k), jnp.bfloat16),
            pltpu.SemaphoreType.DMA((N_DEV - 1,)),
            pltpu.SemaphoreType.DMA((N_DEV - 1,)),
        ],
        compiler_params=pltpu.CompilerParams(collective_id=0),
    )(x, w_mat)


# device time: 33081 ns/iter; 1.7221x vs baseline; 1.7221x over previous
import jax
import jax.numpy as jnp
from jax import lax
from jax.experimental import pallas as pl
from jax.experimental.pallas import tpu as pltpu

N_DEV = 16
R_HOPS = N_DEV // 2
L_HOPS = N_DEV - 1 - R_HOPS


def kernel(x, w_mat):
    m_per, k = x.shape
    _, n_per = w_mat.shape

    def body(x_ref, w_ref, out_ref, ring_r, ring_l,
             r_send, r_recv, l_send, l_recv):
        my = lax.axis_index("i")
        left = lax.rem(my + (N_DEV - 1), N_DEV)
        right = lax.rem(my + 1, N_DEV)

        barrier_sem = pltpu.get_barrier_semaphore()
        for nbr in (left, right):
            pl.semaphore_signal(
                barrier_sem, inc=1,
                device_id=(nbr,), device_id_type=pl.DeviceIdType.MESH,
            )
        pl.semaphore_wait(barrier_sem, 2)

        descs_r = [
            pltpu.make_async_remote_copy(
                src_ref=ring_r.at[s],
                dst_ref=ring_r.at[s + 1],
                send_sem=r_send.at[s],
                recv_sem=r_recv.at[s],
                device_id=(right,),
                device_id_type=pl.DeviceIdType.MESH,
            )
            for s in range(R_HOPS)
        ]
        descs_l = [
            pltpu.make_async_remote_copy(
                src_ref=(ring_r.at[0] if s == 0 else ring_l.at[s]),
                dst_ref=ring_l.at[s + 1],
                send_sem=l_send.at[s],
                recv_sem=l_recv.at[s],
                device_id=(left,),
                device_id_type=pl.DeviceIdType.MESH,
            )
            for s in range(L_HOPS)
        ]

        ring_r[0] = x_ref[...].astype(ring_r.dtype)
        descs_r[0].start()
        descs_l[0].start()

        w_bf16 = w_ref[...].astype(jnp.bfloat16)

        def store_block(origin, chunk):
            y = jnp.dot(chunk, w_bf16, preferred_element_type=jnp.float32)
            out_ref[pl.ds(origin * m_per, m_per), :] = jnp.maximum(y, 0.0)

        store_block(my, ring_r[0])

        for s in range(R_HOPS):
            descs_r[s].wait_recv()
            if s + 1 < R_HOPS:
                descs_r[s + 1].start()
            if s < L_HOPS:
                descs_l[s].wait_recv()
                if s + 1 < L_HOPS:
                    descs_l[s + 1].start()
            store_block(lax.rem(my + (N_DEV - 1 - s), N_DEV), ring_r[s + 1])
            if s < L_HOPS:
                store_block(lax.rem(my + s + 1, N_DEV), ring_l[s + 1])

        for d in descs_r + descs_l:
            d.wait_send()

    return pl.pallas_call(
        body,
        out_shape=jax.ShapeDtypeStruct((N_DEV * m_per, n_per), jnp.float32),
        in_specs=[
            pl.BlockSpec(memory_space=pltpu.VMEM),
            pl.BlockSpec(memory_space=pltpu.VMEM),
        ],
        out_specs=pl.BlockSpec(memory_space=pltpu.VMEM),
        scratch_shapes=[
            pltpu.VMEM((R_HOPS + 1, m_per, k), jnp.bfloat16),
            pltpu.VMEM((L_HOPS + 1, m_per, k), jnp.bfloat16),
            pltpu.SemaphoreType.DMA((R_HOPS,)),
            pltpu.SemaphoreType.DMA((R_HOPS,)),
            pltpu.SemaphoreType.DMA((L_HOPS,)),
            pltpu.SemaphoreType.DMA((L_HOPS,)),
        ],
        compiler_params=pltpu.CompilerParams(collective_id=0),
    )(x, w_mat)


# device time: 30635 ns/iter; 1.8596x vs baseline; 1.0798x over previous
import jax
import jax.numpy as jnp
from jax import lax
from jax.experimental import pallas as pl
from jax.experimental.pallas import tpu as pltpu

N_DEV = 16
R_HOPS = N_DEV // 2
L_HOPS = N_DEV - 1 - R_HOPS

RING = [0, 4, 8, 12, 13, 9, 5, 1, 2, 6, 10, 14, 15, 11, 7, 3]
IDX = [0, 7, 8, 15, 1, 6, 9, 14, 2, 5, 10, 13, 3, 4, 11, 12]


def _lut(table, idx):
    val = jnp.int32(table[0])
    for j in range(1, len(table)):
        val = jnp.where(idx == j, jnp.int32(table[j]), val)
    return val


def kernel(x, w_mat):
    m_per, k = x.shape
    _, n_per = w_mat.shape

    def body(x_ref, w_ref, out_ref, ring_r, ring_l,
             r_send, r_recv, l_send, l_recv):
        my = lax.axis_index("i")
        rpos = _lut(IDX, my)
        right = _lut(RING, lax.rem(rpos + 1, N_DEV))
        left = _lut(RING, lax.rem(rpos + (N_DEV - 1), N_DEV))

        barrier_sem = pltpu.get_barrier_semaphore()
        for nbr in (left, right):
            pl.semaphore_signal(
                barrier_sem, inc=1,
                device_id=(nbr,), device_id_type=pl.DeviceIdType.MESH,
            )
        pl.semaphore_wait(barrier_sem, 2)

        descs_r = [
            pltpu.make_async_remote_copy(
                src_ref=ring_r.at[s],
                dst_ref=ring_r.at[s + 1],
                send_sem=r_send.at[s],
                recv_sem=r_recv.at[s],
                device_id=(right,),
                device_id_type=pl.DeviceIdType.MESH,
            )
            for s in range(R_HOPS)
        ]
        descs_l = [
            pltpu.make_async_remote_copy(
                src_ref=(ring_r.at[0] if s == 0 else ring_l.at[s]),
                dst_ref=ring_l.at[s + 1],
                send_sem=l_send.at[s],
                recv_sem=l_recv.at[s],
                device_id=(left,),
                device_id_type=pl.DeviceIdType.MESH,
            )
            for s in range(L_HOPS)
        ]

        ring_r[0] = x_ref[...].astype(ring_r.dtype)
        descs_r[0].start()
        descs_l[0].start()

        w_bf16 = w_ref[...].astype(jnp.bfloat16)

        def store_block(origin, chunk):
            y = jnp.dot(chunk, w_bf16, preferred_element_type=jnp.float32)
            out_ref[pl.ds(origin * m_per, m_per), :] = jnp.maximum(y, 0.0)

        store_block(my, ring_r[0])

        for s in range(R_HOPS):
            descs_r[s].wait_recv()
            if s + 1 < R_HOPS:
                descs_r[s + 1].start()
            if s < L_HOPS:
                descs_l[s].wait_recv()
                if s + 1 < L_HOPS:
                    descs_l[s + 1].start()
            origin_r = _lut(RING, lax.rem(rpos + (N_DEV - 1 - s), N_DEV))
            store_block(origin_r, ring_r[s + 1])
            if s < L_HOPS:
                origin_l = _lut(RING, lax.rem(rpos + s + 1, N_DEV))
                store_block(origin_l, ring_l[s + 1])

        for d in descs_r + descs_l:
            d.wait_send()

    return pl.pallas_call(
        body,
        out_shape=jax.ShapeDtypeStruct((N_DEV * m_per, n_per), jnp.float32),
        in_specs=[
            pl.BlockSpec(memory_space=pltpu.VMEM),
            pl.BlockSpec(memory_space=pltpu.VMEM),
        ],
        out_specs=pl.BlockSpec(memory_space=pltpu.VMEM),
        scratch_shapes=[
            pltpu.VMEM((R_HOPS + 1, m_per, k), jnp.bfloat16),
            pltpu.VMEM((L_HOPS + 1, m_per, k), jnp.bfloat16),
            pltpu.SemaphoreType.DMA((R_HOPS,)),
            pltpu.SemaphoreType.DMA((R_HOPS,)),
            pltpu.SemaphoreType.DMA((L_HOPS,)),
            pltpu.SemaphoreType.DMA((L_HOPS,)),
        ],
        compiler_params=pltpu.CompilerParams(collective_id=0),
    )(x, w_mat)


# device time: 29962 ns/iter; 1.9014x vs baseline; 1.0225x over previous
import jax
import jax.numpy as jnp
from jax import lax
from jax.experimental import pallas as pl
from jax.experimental.pallas import tpu as pltpu

N_DEV = 16
HOPS = 4

DIRECT_SEND_OFFS = [5, 6, 7, 8, -5, -6, -7]
DIRECT_RECV_OFFS = [-o for o in DIRECT_SEND_OFFS]
N_DIRECT = len(DIRECT_SEND_OFFS)

RING = [0, 4, 8, 12, 13, 9, 5, 1, 2, 6, 10, 14, 15, 11, 7, 3]
IDX = [0, 7, 8, 15, 1, 6, 9, 14, 2, 5, 10, 13, 3, 4, 11, 12]


def _lut(table, idx):
    val = jnp.int32(table[0])
    for j in range(1, len(table)):
        val = jnp.where(idx == j, jnp.int32(table[j]), val)
    return val


def kernel(x, w_mat):
    m_per, k = x.shape
    _, n_per = w_mat.shape

    def body(x_ref, w_ref, out_ref, ring_r, ring_l, dbuf,
             r_send, r_recv, l_send, l_recv, d_send, d_recv):
        my = lax.axis_index("i")
        rpos = _lut(IDX, my)
        right = _lut(RING, lax.rem(rpos + 1, N_DEV))
        left = _lut(RING, lax.rem(rpos + (N_DEV - 1), N_DEV))

        def at_off(off):
            return _lut(RING, lax.rem(rpos + off + N_DEV, N_DEV))

        direct_targets = [at_off(o) for o in DIRECT_SEND_OFFS]

        barrier_sem = pltpu.get_barrier_semaphore()
        for peer in [left, right] + direct_targets:
            pl.semaphore_signal(
                barrier_sem, inc=1,
                device_id=(peer,), device_id_type=pl.DeviceIdType.MESH,
            )
        pl.semaphore_wait(barrier_sem, 2 + N_DIRECT)

        descs_r = [
            pltpu.make_async_remote_copy(
                src_ref=ring_r.at[s],
                dst_ref=ring_r.at[s + 1],
                send_sem=r_send.at[s],
                recv_sem=r_recv.at[s],
                device_id=(right,),
                device_id_type=pl.DeviceIdType.MESH,
            )
            for s in range(HOPS)
        ]
        descs_l = [
            pltpu.make_async_remote_copy(
                src_ref=(ring_r.at[0] if s == 0 else ring_l.at[s]),
                dst_ref=ring_l.at[s + 1],
                send_sem=l_send.at[s],
                recv_sem=l_recv.at[s],
                device_id=(left,),
                device_id_type=pl.DeviceIdType.MESH,
            )
            for s in range(HOPS)
        ]
        descs_d = [
            pltpu.make_async_remote_copy(
                src_ref=ring_r.at[0],
                dst_ref=dbuf.at[j],
                send_sem=d_send.at[j],
                recv_sem=d_recv.at[j],
                device_id=(direct_targets[j],),
                device_id_type=pl.DeviceIdType.MESH,
            )
            for j in range(N_DIRECT)
        ]

        ring_r[0] = x_ref[...].astype(ring_r.dtype)
        for d in descs_d:
            d.start()
        descs_r[0].start()
        descs_l[0].start()

        w_bf16 = w_ref[...].astype(jnp.bfloat16)

        def store_block(origin, chunk):
            y = jnp.dot(chunk, w_bf16, preferred_element_type=jnp.float32)
            out_ref[pl.ds(origin * m_per, m_per), :] = jnp.maximum(y, 0.0)

        store_block(my, ring_r[0])

        for s in range(HOPS):
            descs_r[s].wait_recv()
            if s + 1 < HOPS:
                descs_r[s + 1].start()
            descs_l[s].wait_recv()
            if s + 1 < HOPS:
                descs_l[s + 1].start()
            store_block(at_off(-(s + 1)), ring_r[s + 1])
            store_block(at_off(s + 1), ring_l[s + 1])

        for j in range(N_DIRECT):
            descs_d[j].wait_recv()
            store_block(at_off(DIRECT_RECV_OFFS[j]), dbuf[j])

        for d in descs_r + descs_l + descs_d:
            d.wait_send()

    return pl.pallas_call(
        body,
        out_shape=jax.ShapeDtypeStruct((N_DEV * m_per, n_per), jnp.float32),
        in_specs=[
            pl.BlockSpec(memory_space=pltpu.VMEM),
            pl.BlockSpec(memory_space=pltpu.VMEM),
        ],
        out_specs=pl.BlockSpec(memory_space=pltpu.VMEM),
        scratch_shapes=[
            pltpu.VMEM((HOPS + 1, m_per, k), jnp.bfloat16),
            pltpu.VMEM((HOPS + 1, m_per, k), jnp.bfloat16),
            pltpu.VMEM((N_DIRECT, m_per, k), jnp.bfloat16),
            pltpu.SemaphoreType.DMA((HOPS,)),
            pltpu.SemaphoreType.DMA((HOPS,)),
            pltpu.SemaphoreType.DMA((HOPS,)),
            pltpu.SemaphoreType.DMA((HOPS,)),
            pltpu.SemaphoreType.DMA((N_DIRECT,)),
            pltpu.SemaphoreType.DMA((N_DIRECT,)),
        ],
        compiler_params=pltpu.CompilerParams(collective_id=0),
    )(x, w_mat)


# device time: 25586 ns/iter; 2.2266x vs baseline; 1.1710x over previous
import jax
import jax.numpy as jnp
from jax import lax
from jax.experimental import pallas as pl
from jax.experimental.pallas import tpu as pltpu

N_DEV = 16

DQ = [0, 3, 1, 2]


def kernel(x, w_mat):
    m_per, k = x.shape
    _, n_per = w_mat.shape

    def body(x_ref, w_ref, out_ref, P, Z,
             p_send, p_recv, zu_send, zd_send, f_send, z_recv):
        my = lax.axis_index("i")
        q = lax.rem(my, 4)
        z = my // 4
        q_up = 4 * z + lax.rem(q + 1, 4)
        q_dn = 4 * z + lax.rem(q + 3, 4)
        z_up = lax.rem(my + 4, N_DEV)
        z_dn = lax.rem(my + 12, N_DEV)

        barrier_sem = pltpu.get_barrier_semaphore()
        for peer in (q_up, q_dn, z_up, z_dn):
            pl.semaphore_signal(
                barrier_sem, inc=1,
                device_id=(peer,), device_id_type=pl.DeviceIdType.MESH,
            )
        pl.semaphore_wait(barrier_sem, 4)

        def rdma(src, dst, ssem, rsem, dev):
            return pltpu.make_async_remote_copy(
                src_ref=src, dst_ref=dst, send_sem=ssem, recv_sem=rsem,
                device_id=(dev,), device_id_type=pl.DeviceIdType.MESH,
            )

        d_p_up = rdma(P.at[0], P.at[1], p_send.at[0], p_recv.at[0], q_up)
        d_p_dn = rdma(P.at[0], P.at[2], p_send.at[1], p_recv.at[1], q_dn)
        d_p_fwd = rdma(P.at[1], P.at[3], p_send.at[2], p_recv.at[2], q_up)
        d_z_up = [
            rdma(P.at[j], Z.at[j, 0], zu_send.at[j], z_recv.at[j, 0], z_up)
            for j in range(4)
        ]
        d_z_dn = [
            rdma(P.at[j], Z.at[j, 1], zd_send.at[j], z_recv.at[j, 1], z_dn)
            for j in range(4)
        ]
        d_z_fwd = [
            rdma(Z.at[j, 0] if j < 2 else Z.at[j, 1], Z.at[j, 2],
                 f_send.at[j], z_recv.at[j, 2], z_up if j < 2 else z_dn)
            for j in range(4)
        ]

        w_bf16 = w_ref[...].astype(jnp.bfloat16)

        def store_block(origin, chunk):
            y = jnp.dot(chunk, w_bf16, preferred_element_type=jnp.float32)
            out_ref[pl.ds(origin * m_per, m_per), :] = jnp.maximum(y, 0.0)

        def orig(dz, j):
            return 4 * lax.rem(z + dz, 4) + lax.rem(q + DQ[j], 4)

        P[0] = x_ref[...].astype(P.dtype)
        d_p_up.start()
        d_p_dn.start()
        d_z_up[0].start()
        d_z_dn[0].start()
        store_block(my, P[0])

        d_p_up.wait_recv()
        d_p_fwd.start()
        d_z_up[1].start()
        d_z_dn[1].start()
        store_block(orig(0, 1), P[1])

        d_p_dn.wait_recv()
        d_z_up[2].start()
        d_z_dn[2].start()
        store_block(orig(0, 2), P[2])

        d_p_fwd.wait_recv()
        d_z_up[3].start()
        d_z_dn[3].start()
        store_block(orig(0, 3), P[3])

        for j in range(4):
            d_z_up[j].wait_recv()
            if j < 2:
                d_z_fwd[j].start()
            store_block(orig(3, j), Z[j, 0])
            d_z_dn[j].wait_recv()
            if j >= 2:
                d_z_fwd[j].start()
            store_block(orig(1, j), Z[j, 1])

        for j in range(4):
            d_z_fwd[j].wait_recv()
            store_block(orig(2, j), Z[j, 2])

        for d in [d_p_up, d_p_dn, d_p_fwd] + d_z_up + d_z_dn + d_z_fwd:
            d.wait_send()

    return pl.pallas_call(
        body,
        out_shape=jax.ShapeDtypeStruct((N_DEV * m_per, n_per), jnp.float32),
        in_specs=[
            pl.BlockSpec(memory_space=pltpu.VMEM),
            pl.BlockSpec(memory_space=pltpu.VMEM),
        ],
        out_specs=pl.BlockSpec(memory_space=pltpu.VMEM),
        scratch_shapes=[
            pltpu.VMEM((4, m_per, k), jnp.bfloat16),
            pltpu.VMEM((4, 3, m_per, k), jnp.bfloat16),
            pltpu.SemaphoreType.DMA((3,)),
            pltpu.SemaphoreType.DMA((3,)),
            pltpu.SemaphoreType.DMA((4,)),
            pltpu.SemaphoreType.DMA((4,)),
            pltpu.SemaphoreType.DMA((4,)),
            pltpu.SemaphoreType.DMA((4, 3)),
        ],
        compiler_params=pltpu.CompilerParams(collective_id=0),
    )(x, w_mat)
